# baseline (speedup 1.0000x reference)
import functools

import jax
import jax.numpy as jnp
from jax import lax
from jax.experimental import pallas as pl
from jax.experimental.pallas import tpu as pltpu
from jax.experimental.pallas import tpu_sc as plsc

N_NODES = 10000
N_EDGES = 320000
D_IN = 128
D_OUT = 64

NC, NS, L = 2, 16, 16
NW = NC * NS
CHUNK = 128
CPT = 2 * (-(-N_EDGES // (NW * CHUNK * 2)))
E_PAD = NW * CPT * CHUNK
ROW_BLK = 512
N_PAD = ROW_BLK * (-(-(N_NODES + 1) // ROW_BLK))
RPT = N_PAD // NS


def _deg_body(dst_hbm, z_hbm, degp_hbm, dst_all, ones_v, deg_sh, ssem):
    c = lax.axis_index("c")
    s = lax.axis_index("s")
    wid = c * NS + s
    row0 = s * RPT

    pltpu.sync_copy(dst_hbm.at[wid], dst_all)
    pltpu.sync_copy(z_hbm.at[pl.ds(row0, RPT)], deg_sh.at[pl.ds(row0, RPT)])
    for i in range(CHUNK // L):
        ones_v[pl.ds(i * L, L)] = jnp.ones((L,), jnp.float32)
    plsc.subcore_barrier()

    def body(j, carry):
        pltpu.async_copy(ones_v, deg_sh.at[dst_all.at[j]], ssem, add=True)
        return carry

    lax.fori_loop(0, CPT, body, 0)

    def drain(j, carry):
        pltpu.make_async_copy(ones_v, deg_sh.at[dst_all.at[0]], ssem).wait()
        return carry

    lax.fori_loop(0, CPT, drain, 0)
    plsc.subcore_barrier()
    pltpu.sync_copy(deg_sh.at[pl.ds(row0, RPT)],
                    degp_hbm.at[c, pl.ds(row0, RPT)])


def _scatter_body(src_hbm, dst_hbm, h2_hbm, z_hbm, accp_hbm,
                  src_all, dst_all, rows0, rows1, acc_sh, gsem0, gsem1):
    c = lax.axis_index("c")
    s = lax.axis_index("s")
    wid = c * NS + s
    row0 = s * RPT

    pltpu.sync_copy(src_hbm.at[wid], src_all)
    pltpu.sync_copy(dst_hbm.at[wid], dst_all)

    @pl.when(c == 0)
    def _():
        pltpu.sync_copy(h2_hbm.at[pl.ds(row0, RPT)], acc_sh.at[pl.ds(row0, RPT)])

    @pl.when(c == 1)
    def _():
        pltpu.sync_copy(z_hbm.at[pl.ds(row0, RPT)], acc_sh.at[pl.ds(row0, RPT)])

    plsc.subcore_barrier()

    rows = (rows0, rows1)
    gsem = (gsem0, gsem1)

    def gather(j, b):
        return pltpu.async_copy(h2_hbm.at[src_all.at[j]], rows[b], gsem[b])

    def scatter(j, b):
        pltpu.sync_copy(rows[b], acc_sh.at[dst_all.at[j]], add=True)

    gather(0, 0)

    def body(i, carry):
        j = 2 * i
        gather(j + 1, 1)
        pltpu.make_async_copy(h2_hbm.at[src_all.at[j]], rows[0], gsem[0]).wait()
        scatter(j, 0)
        gather(j + 2, 0)
        pltpu.make_async_copy(h2_hbm.at[src_all.at[j]], rows[1], gsem[1]).wait()
        scatter(j + 1, 1)
        return carry

    lax.fori_loop(0, CPT // 2 - 1, body, 0)

    j = CPT - 2
    gather(j + 1, 1)
    pltpu.make_async_copy(h2_hbm.at[src_all.at[j]], rows[0], gsem[0]).wait()
    scatter(j, 0)
    pltpu.make_async_copy(h2_hbm.at[src_all.at[j]], rows[1], gsem[1]).wait()
    scatter(j + 1, 1)

    plsc.subcore_barrier()
    pltpu.sync_copy(acc_sh.at[pl.ds(row0, RPT)],
                    accp_hbm.at[c, pl.ds(row0, RPT)])


def _h2_tc_body(x_ref, w_ref, degp_ref, h2_ref, dis_ref):
    deg = degp_ref[0, :] + degp_ref[1, :] + 1.0
    dis = lax.rsqrt(deg)
    h = jnp.dot(x_ref[...], w_ref[...], preferred_element_type=jnp.float32)
    h2_ref[...] = h * dis[:, None]
    dis_ref[...] = dis[:, None]


def _final_tc_body(accp_ref, dis_ref, b_ref, out_ref):
    a = accp_ref[0] + accp_ref[1]
    out_ref[...] = a * dis_ref[...] + b_ref[...]


def kernel(x, edge_index, W, b):
    src = edge_index[0].astype(jnp.int32)
    dst = edge_index[1].astype(jnp.int32)
    pad = E_PAD - N_EDGES
    src_p = jnp.concatenate([src, jnp.zeros((pad,), jnp.int32)])
    dst_p = jnp.concatenate([dst, jnp.full((pad,), N_NODES, jnp.int32)])
    src_p = src_p.reshape(NW, CPT, CHUNK)
    dst_p = dst_p.reshape(NW, CPT, CHUNK)
    x_p = jnp.pad(x, ((0, N_PAD - N_NODES), (0, 0)))
    z_row = jnp.zeros((N_PAD,), jnp.float32)
    z_mat = jnp.zeros((N_PAD, D_OUT), jnp.float32)

    mesh = plsc.VectorSubcoreMesh(core_axis_name="c", subcore_axis_name="s")

    deg_k = functools.partial(
        pl.kernel,
        out_type=jax.ShapeDtypeStruct((NC, N_PAD), jnp.float32),
        mesh=mesh,
        scratch_types=[
            pltpu.VMEM((CPT, CHUNK), jnp.int32),
            pltpu.VMEM((CHUNK,), jnp.float32),
            pltpu.VMEM_SHARED((N_PAD,), jnp.float32),
            pltpu.SemaphoreType.DMA,
        ],
    )(_deg_body)
    degp = deg_k(dst_p, z_row)

    n_blocks = N_PAD // ROW_BLK
    h2, dis = pl.pallas_call(
        _h2_tc_body,
        grid=(n_blocks,),
        in_specs=[
            pl.BlockSpec((ROW_BLK, D_IN), lambda i: (i, 0)),
            pl.BlockSpec((D_IN, D_OUT), lambda i: (0, 0)),
            pl.BlockSpec((NC, ROW_BLK), lambda i: (0, i)),
        ],
        out_specs=[
            pl.BlockSpec((ROW_BLK, D_OUT), lambda i: (i, 0)),
            pl.BlockSpec((ROW_BLK, 1), lambda i: (i, 0)),
        ],
        out_shape=[
            jax.ShapeDtypeStruct((N_PAD, D_OUT), jnp.float32),
            jax.ShapeDtypeStruct((N_PAD, 1), jnp.float32),
        ],
    )(x_p, W, degp)

    scat_k = functools.partial(
        pl.kernel,
        out_type=jax.ShapeDtypeStruct((NC, N_PAD, D_OUT), jnp.float32),
        mesh=mesh,
        compiler_params=pltpu.CompilerParams(use_tc_tiling_on_sc=False),
        scratch_types=[
            pltpu.VMEM((CPT, CHUNK), jnp.int32),
            pltpu.VMEM((CPT, CHUNK), jnp.int32),
            pltpu.VMEM((CHUNK, D_OUT), jnp.float32),
            pltpu.VMEM((CHUNK, D_OUT), jnp.float32),
            pltpu.VMEM_SHARED((N_PAD, D_OUT), jnp.float32),
            pltpu.SemaphoreType.DMA,
            pltpu.SemaphoreType.DMA,
        ],
    )(_scatter_body)
    accp = scat_k(src_p, dst_p, h2, z_mat)

    out = pl.pallas_call(
        _final_tc_body,
        grid=(n_blocks,),
        in_specs=[
            pl.BlockSpec((NC, ROW_BLK, D_OUT), lambda i: (0, i, 0)),
            pl.BlockSpec((ROW_BLK, 1), lambda i: (i, 0)),
            pl.BlockSpec((1, D_OUT), lambda i: (0, 0)),
        ],
        out_specs=pl.BlockSpec((ROW_BLK, D_OUT), lambda i: (i, 0)),
        out_shape=jax.ShapeDtypeStruct((N_PAD, D_OUT), jnp.float32),
    )(accp, dis, b.reshape(1, D_OUT))

    return out[:N_NODES]


# speedup vs baseline: 2.3268x; 2.3268x over previous
import functools

import jax
import jax.numpy as jnp
from jax import lax
from jax.experimental import pallas as pl
from jax.experimental.pallas import tpu as pltpu
from jax.experimental.pallas import tpu_sc as plsc

N_NODES = 10000
N_EDGES = 320000
D_IN = 128
D_OUT = 64

NC, NS, L = 2, 16, 16
NW = NC * NS
CHUNK = 128
CPT = -(-N_EDGES // (NW * CHUNK))
E_PAD = NW * CPT * CHUNK
ROW_BLK = 512
N_PAD = ROW_BLK * (-(-(N_NODES + 1) // ROW_BLK))
RPT = N_PAD // NS


def _deg_body(ei_hbm, degp_hbm, dst_all, ones_v, zcol_v, deg_sh, ssem):
    c = lax.axis_index("c")
    s = lax.axis_index("s")
    wid = c * NS + s
    row0 = s * RPT

    pltpu.sync_copy(ei_hbm.at[1, wid], dst_all)
    for i in range(CHUNK // L):
        ones_v[pl.ds(i * L, L)] = jnp.ones((L,), jnp.float32)

    def zero(i, carry):
        zcol_v[pl.ds(i * L, L)] = jnp.zeros((L,), jnp.float32)
        return carry

    lax.fori_loop(0, RPT // L, zero, 0)
    pltpu.sync_copy(zcol_v, deg_sh.at[pl.ds(row0, RPT)])
    plsc.subcore_barrier()

    def body(j, carry):
        pltpu.async_copy(ones_v, deg_sh.at[dst_all.at[j]], ssem, add=True)
        return carry

    lax.fori_loop(0, CPT, body, 0)

    def drain(j, carry):
        pltpu.make_async_copy(ones_v, deg_sh.at[dst_all.at[0]], ssem).wait()
        return carry

    lax.fori_loop(0, CPT, drain, 0)
    plsc.subcore_barrier()
    pltpu.sync_copy(deg_sh.at[pl.ds(row0, RPT)],
                    degp_hbm.at[c, pl.ds(row0, RPT)])


def _scatter_body(ei_hbm, h2_hbm, accp_hbm,
                  src_all, dst_all, rows0, rows1, h2_sh, acc_sh,
                  gsem0, gsem1):
    c = lax.axis_index("c")
    s = lax.axis_index("s")
    wid = c * NS + s
    row0 = s * RPT

    pltpu.sync_copy(ei_hbm.at[0, wid], src_all)
    pltpu.sync_copy(ei_hbm.at[1, wid], dst_all)
    pltpu.sync_copy(h2_hbm.at[pl.ds(row0, RPT)], h2_sh.at[pl.ds(row0, RPT)])

    @pl.when(c == 0)
    def _():
        pltpu.sync_copy(h2_hbm.at[pl.ds(row0, RPT)], acc_sh.at[pl.ds(row0, RPT)])

    @pl.when(c == 1)
    def _():
        def zero(i, carry):
            for k in range(D_OUT // L):
                rows0[i, pl.ds(k * L, L)] = jnp.zeros((L,), jnp.float32)
            return carry

        lax.fori_loop(0, CHUNK, zero, 0)
        for j in range(RPT // CHUNK):
            pltpu.sync_copy(rows0, acc_sh.at[pl.ds(row0 + j * CHUNK, CHUNK)])

    plsc.subcore_barrier()

    rows = (rows0, rows1)
    gsem = (gsem0, gsem1)

    def gather(j, b):
        pltpu.async_copy(h2_sh.at[src_all.at[j]], rows[b], gsem[b])

    def gwait(b):
        pltpu.make_async_copy(h2_sh.at[src_all.at[0]], rows[b], gsem[b]).wait()

    def scatter(j, b):
        pltpu.sync_copy(rows[b], acc_sh.at[dst_all.at[j]], add=True)

    gather(0, 0)

    def body(i, carry):
        j = 2 * i
        gather(j + 1, 1)
        gwait(0)
        scatter(j, 0)
        gather(j + 2, 0)
        gwait(1)
        scatter(j + 1, 1)
        return carry

    lax.fori_loop(0, (CPT - 1) // 2, body, 0)
    gwait(0)
    scatter(CPT - 1, 0)

    plsc.subcore_barrier()
    pltpu.sync_copy(acc_sh.at[pl.ds(row0, RPT)],
                    accp_hbm.at[c, pl.ds(row0, RPT)])


def _h2_tc_body(x_ref, w_ref, degp_ref, h2_ref, dis_ref):
    deg = degp_ref[0, :] + degp_ref[1, :] + 1.0
    dis = lax.rsqrt(deg)
    h = jnp.dot(x_ref[...], w_ref[...], preferred_element_type=jnp.float32)
    h2_ref[...] = h * dis[:, None]
    dis_ref[...] = dis[:, None]


def _final_tc_body(accp_ref, dis_ref, b_ref, out_ref):
    a = accp_ref[0] + accp_ref[1]
    out_ref[...] = a * dis_ref[...] + b_ref[...]


def kernel(x, edge_index, W, b):
    pad = E_PAD - N_EDGES
    pad_block = jnp.broadcast_to(
        jnp.array([[0], [N_NODES]], jnp.int32), (2, pad))
    ei_p = jnp.concatenate([edge_index.astype(jnp.int32), pad_block], axis=1)
    ei_p = ei_p.reshape(2, NW, CPT, CHUNK)

    mesh = plsc.VectorSubcoreMesh(core_axis_name="c", subcore_axis_name="s")

    deg_k = functools.partial(
        pl.kernel,
        out_type=jax.ShapeDtypeStruct((NC, N_PAD), jnp.float32),
        mesh=mesh,
        scratch_types=[
            pltpu.VMEM((CPT, CHUNK), jnp.int32),
            pltpu.VMEM((CHUNK,), jnp.float32),
            pltpu.VMEM((RPT,), jnp.float32),
            pltpu.VMEM_SHARED((N_PAD,), jnp.float32),
            pltpu.SemaphoreType.DMA,
        ],
    )(_deg_body)
    degp = deg_k(ei_p)

    n_blocks = N_PAD // ROW_BLK
    h2, dis = pl.pallas_call(
        _h2_tc_body,
        grid=(n_blocks,),
        in_specs=[
            pl.BlockSpec((ROW_BLK, D_IN), lambda i: (i, 0)),
            pl.BlockSpec((D_IN, D_OUT), lambda i: (0, 0)),
            pl.BlockSpec((NC, ROW_BLK), lambda i: (0, i)),
        ],
        out_specs=[
            pl.BlockSpec((ROW_BLK, D_OUT), lambda i: (i, 0)),
            pl.BlockSpec((ROW_BLK, 1), lambda i: (i, 0)),
        ],
        out_shape=[
            jax.ShapeDtypeStruct((N_PAD, D_OUT), jnp.float32),
            jax.ShapeDtypeStruct((N_PAD, 1), jnp.float32),
        ],
    )(x, W, degp)

    scat_k = functools.partial(
        pl.kernel,
        out_type=jax.ShapeDtypeStruct((NC, N_PAD, D_OUT), jnp.float32),
        mesh=mesh,
        compiler_params=pltpu.CompilerParams(use_tc_tiling_on_sc=False),
        scratch_types=[
            pltpu.VMEM((CPT, CHUNK), jnp.int32),
            pltpu.VMEM((CPT, CHUNK), jnp.int32),
            pltpu.VMEM((CHUNK, D_OUT), jnp.float32),
            pltpu.VMEM((CHUNK, D_OUT), jnp.float32),
            pltpu.VMEM_SHARED((N_PAD, D_OUT), jnp.float32),
            pltpu.VMEM_SHARED((N_PAD, D_OUT), jnp.float32),
            pltpu.SemaphoreType.DMA,
            pltpu.SemaphoreType.DMA,
        ],
    )(_scatter_body)
    accp = scat_k(ei_p, h2)

    out = pl.pallas_call(
        _final_tc_body,
        grid=(n_blocks,),
        in_specs=[
            pl.BlockSpec((NC, ROW_BLK, D_OUT), lambda i: (0, i, 0)),
            pl.BlockSpec((ROW_BLK, 1), lambda i: (i, 0)),
            pl.BlockSpec((1, D_OUT), lambda i: (0, 0)),
        ],
        out_specs=pl.BlockSpec((ROW_BLK, D_OUT), lambda i: (i, 0)),
        out_shape=jax.ShapeDtypeStruct((N_NODES, D_OUT), jnp.float32),
    )(accp, dis, b.reshape(1, D_OUT))

    return out


# speedup vs baseline: 2.3481x; 1.0092x over previous
import functools

import jax
import jax.numpy as jnp
from jax import lax
from jax.experimental import pallas as pl
from jax.experimental.pallas import tpu as pltpu
from jax.experimental.pallas import tpu_sc as plsc

N_NODES = 10000
N_EDGES = 320000
D_IN = 128
D_OUT = 64

NC, NS, L = 2, 16, 16
NW = NC * NS
CHUNK = 128
CPT = 2 * (-(-N_EDGES // (NW * CHUNK * 2)))
E_PAD = NW * CPT * CHUNK
ROW_BLK = 512
N_PAD = ROW_BLK * (-(-(N_NODES + 1) // ROW_BLK))
RPT = N_PAD // NS


def _deg_body(dst_hbm, degp_hbm, dst_all, ones_v, zcol_v, deg_sh, ssem):
    c = lax.axis_index("c")
    s = lax.axis_index("s")
    wid = c * NS + s
    row0 = s * RPT

    pltpu.sync_copy(dst_hbm.at[wid], dst_all)
    for i in range(CHUNK // L):
        ones_v[pl.ds(i * L, L)] = jnp.ones((L,), jnp.float32)

    def zero(i, carry):
        zcol_v[pl.ds(i * L, L)] = jnp.zeros((L,), jnp.float32)
        return carry

    lax.fori_loop(0, RPT // L, zero, 0)
    pltpu.sync_copy(zcol_v, deg_sh.at[pl.ds(row0, RPT)])
    plsc.subcore_barrier()

    def body(j, carry):
        pltpu.async_copy(ones_v, deg_sh.at[dst_all.at[j]], ssem, add=True)
        return carry

    lax.fori_loop(0, CPT, body, 0)

    def drain(j, carry):
        pltpu.make_async_copy(ones_v, deg_sh.at[dst_all.at[0]], ssem).wait()
        return carry

    lax.fori_loop(0, CPT, drain, 0)
    plsc.subcore_barrier()
    pltpu.sync_copy(deg_sh.at[pl.ds(row0, RPT)],
                    degp_hbm.at[pl.ds(c * N_PAD + row0, RPT)])


def _scatter_body(src_hbm, dst_hbm, h2_hbm, accp_hbm,
                  src_all, dst_all, rows0, rows1, h2_sh, acc_sh,
                  gsem0, gsem1):
    c = lax.axis_index("c")
    s = lax.axis_index("s")
    wid = c * NS + s
    row0 = s * RPT

    pltpu.sync_copy(src_hbm.at[wid], src_all)
    pltpu.sync_copy(dst_hbm.at[wid], dst_all)
    pltpu.sync_copy(h2_hbm.at[pl.ds(row0, RPT)], h2_sh.at[pl.ds(row0, RPT)])

    @pl.when(c == 0)
    def _():
        pltpu.sync_copy(h2_hbm.at[pl.ds(row0, RPT)], acc_sh.at[pl.ds(row0, RPT)])

    @pl.when(c == 1)
    def _():
        def zero(i, carry):
            for k in range(D_OUT // L):
                rows0[i, pl.ds(k * L, L)] = jnp.zeros((L,), jnp.float32)
            return carry

        lax.fori_loop(0, CHUNK, zero, 0)
        for j in range(RPT // CHUNK):
            pltpu.sync_copy(rows0, acc_sh.at[pl.ds(row0 + j * CHUNK, CHUNK)])

    plsc.subcore_barrier()

    rows = (rows0, rows1)
    gsem = (gsem0, gsem1)

    def gather(j, b):
        pltpu.async_copy(h2_sh.at[src_all.at[j]], rows[b], gsem[b])

    def gwait(b):
        pltpu.make_async_copy(h2_sh.at[src_all.at[0]], rows[b], gsem[b]).wait()

    def scatter(j, b):
        pltpu.sync_copy(rows[b], acc_sh.at[dst_all.at[j]], add=True)

    gather(0, 0)

    def body(i, carry):
        j = 2 * i
        gather(j + 1, 1)
        gwait(0)
        scatter(j, 0)
        gather(j + 2, 0)
        gwait(1)
        scatter(j + 1, 1)
        return carry

    lax.fori_loop(0, (CPT - 2) // 2, body, 0)
    gather(CPT - 1, 1)
    gwait(0)
    scatter(CPT - 2, 0)
    gwait(1)
    scatter(CPT - 1, 1)

    plsc.subcore_barrier()
    pltpu.sync_copy(acc_sh.at[pl.ds(row0, RPT)],
                    accp_hbm.at[c, pl.ds(row0, RPT)])


def _h2_tc_body(x_ref, w_ref, degp_ref, h2_ref, dis_ref):
    deg = degp_ref[0, :] + degp_ref[1, :] + 1.0
    dis = lax.rsqrt(deg)
    h = jnp.dot(x_ref[...], w_ref[...], preferred_element_type=jnp.float32)
    h2_ref[...] = h * dis[:, None]
    dis_ref[...] = dis


def _final_tc_body(a0_ref, a1_ref, disr_ref, b_ref, out_ref):
    a = a0_ref[0] + a1_ref[0]
    out_ref[...] = a * disr_ref[...] + b_ref[...]


def kernel(x, edge_index, W, b):
    pad = E_PAD - N_EDGES
    src3 = jnp.pad(edge_index[0].astype(jnp.int32), (0, pad))
    src3 = src3.reshape(NW, CPT, CHUNK)
    dst3 = jnp.pad(edge_index[1].astype(jnp.int32), (0, pad),
                   constant_values=N_NODES)
    dst3 = dst3.reshape(NW, CPT, CHUNK)
    b2 = jnp.concatenate([b, b]).reshape(1, 2 * D_OUT)

    mesh = plsc.VectorSubcoreMesh(core_axis_name="c", subcore_axis_name="s")

    deg_k = functools.partial(
        pl.kernel,
        out_type=jax.ShapeDtypeStruct((NC * N_PAD,), jnp.float32),
        mesh=mesh,
        scratch_types=[
            pltpu.VMEM((CPT, CHUNK), jnp.int32),
            pltpu.VMEM((CHUNK,), jnp.float32),
            pltpu.VMEM((RPT,), jnp.float32),
            pltpu.VMEM_SHARED((N_PAD,), jnp.float32),
            pltpu.SemaphoreType.DMA,
        ],
    )(_deg_body)
    degp = deg_k(dst3).reshape(NC, N_PAD)

    n_blocks = N_PAD // ROW_BLK
    h2, dis = pl.pallas_call(
        _h2_tc_body,
        grid=(n_blocks,),
        in_specs=[
            pl.BlockSpec((ROW_BLK, D_IN), lambda i: (i, 0)),
            pl.BlockSpec((D_IN, D_OUT), lambda i: (0, 0)),
            pl.BlockSpec((NC, ROW_BLK), lambda i: (0, i)),
        ],
        out_specs=[
            pl.BlockSpec((ROW_BLK, D_OUT), lambda i: (i, 0)),
            pl.BlockSpec((ROW_BLK,), lambda i: (i,)),
        ],
        out_shape=[
            jax.ShapeDtypeStruct((N_PAD, D_OUT), jnp.float32),
            jax.ShapeDtypeStruct((N_PAD,), jnp.float32),
        ],
    )(x, W, degp)
    disr = jnp.broadcast_to(
        dis.reshape(N_PAD // 2, 2, 1), (N_PAD // 2, 2, D_OUT)
    ).reshape(N_PAD // 2, 2 * D_OUT)

    scat_k = functools.partial(
        pl.kernel,
        out_type=jax.ShapeDtypeStruct((NC, N_PAD, D_OUT), jnp.float32),
        mesh=mesh,
        compiler_params=pltpu.CompilerParams(use_tc_tiling_on_sc=False),
        scratch_types=[
            pltpu.VMEM((CPT, CHUNK), jnp.int32),
            pltpu.VMEM((CPT, CHUNK), jnp.int32),
            pltpu.VMEM((CHUNK, D_OUT), jnp.float32),
            pltpu.VMEM((CHUNK, D_OUT), jnp.float32),
            pltpu.VMEM_SHARED((N_PAD, D_OUT), jnp.float32),
            pltpu.VMEM_SHARED((N_PAD, D_OUT), jnp.float32),
            pltpu.SemaphoreType.DMA,
            pltpu.SemaphoreType.DMA,
        ],
    )(_scatter_body)
    accp = scat_k(src3, dst3, h2)
    ap = accp.reshape(NC, N_PAD // 2, 2 * D_OUT)

    out_blocks = -(-(N_NODES // 2) // (ROW_BLK // 2))
    outp = pl.pallas_call(
        _final_tc_body,
        grid=(out_blocks,),
        in_specs=[
            pl.BlockSpec((1, ROW_BLK // 2, 2 * D_OUT), lambda i: (0, i, 0)),
            pl.BlockSpec((1, ROW_BLK // 2, 2 * D_OUT), lambda i: (1, i, 0)),
            pl.BlockSpec((ROW_BLK // 2, 2 * D_OUT), lambda i: (i, 0)),
            pl.BlockSpec((1, 2 * D_OUT), lambda i: (0, 0)),
        ],
        out_specs=pl.BlockSpec((ROW_BLK // 2, 2 * D_OUT), lambda i: (i, 0)),
        out_shape=jax.ShapeDtypeStruct((N_NODES // 2, 2 * D_OUT), jnp.float32),
    )(ap, ap, disr, b2)

    return outp.reshape(N_NODES, D_OUT)


# speedup vs baseline: 2.6308x; 1.1204x over previous
import functools

import jax
import jax.numpy as jnp
from jax import lax
from jax.experimental import pallas as pl
from jax.experimental.pallas import tpu as pltpu
from jax.experimental.pallas import tpu_sc as plsc

N_NODES = 10000
N_EDGES = 320000
D_IN = 128
D_OUT = 64

NC, NS, L = 2, 16, 16
NW = NC * NS
CHUNK = 128
CPT = 2 * (-(-N_EDGES // (NW * CHUNK * 2)))
E_PAD = NW * CPT * CHUNK
ROW_BLK = 512
N_PAD = ROW_BLK * (-(-(N_NODES + 1) // ROW_BLK))
RPT = N_PAD // NS


def _deg_body(ei_hbm, degp_hbm, dst_all, ones_v, zcol_v, deg_sh, ssem):
    c = lax.axis_index("c")
    s = lax.axis_index("s")
    wid = c * NS + s
    row0 = s * RPT

    pltpu.sync_copy(ei_hbm.at[1, wid], dst_all)
    for i in range(CHUNK // L):
        ones_v[pl.ds(i * L, L)] = jnp.ones((L,), jnp.float32)

    def zero(i, carry):
        zcol_v[pl.ds(i * L, L)] = jnp.zeros((L,), jnp.float32)
        return carry

    lax.fori_loop(0, RPT // L, zero, 0)
    pltpu.sync_copy(zcol_v, deg_sh.at[pl.ds(row0, RPT)])
    plsc.subcore_barrier()

    def body(j, carry):
        pltpu.async_copy(ones_v, deg_sh.at[dst_all.at[j]], ssem, add=True)
        return carry

    lax.fori_loop(0, CPT, body, 0)

    def drain(j, carry):
        pltpu.make_async_copy(ones_v, deg_sh.at[dst_all.at[0]], ssem).wait()
        return carry

    lax.fori_loop(0, CPT, drain, 0)
    plsc.subcore_barrier()
    pltpu.sync_copy(deg_sh.at[pl.ds(row0, RPT)],
                    degp_hbm.at[pl.ds(c * N_PAD + row0, RPT)])


def _scatter_body(ei_hbm, h2_hbm, accp_hbm,
                  src_all, dst_all, rows0, rows1, h2_sh, acc_sh,
                  gsem0, gsem1):
    c = lax.axis_index("c")
    s = lax.axis_index("s")
    wid = c * NS + s
    row0 = s * RPT

    pltpu.sync_copy(ei_hbm.at[0, wid], src_all)
    pltpu.sync_copy(ei_hbm.at[1, wid], dst_all)
    pltpu.sync_copy(h2_hbm.at[pl.ds(row0, RPT)], h2_sh.at[pl.ds(row0, RPT)])

    @pl.when(c == 0)
    def _():
        pltpu.sync_copy(h2_hbm.at[pl.ds(row0, RPT)], acc_sh.at[pl.ds(row0, RPT)])

    @pl.when(c == 1)
    def _():
        def zero(i, carry):
            for k in range(D_OUT // L):
                rows0[i, pl.ds(k * L, L)] = jnp.zeros((L,), jnp.float32)
            return carry

        lax.fori_loop(0, CHUNK, zero, 0)
        for j in range(RPT // CHUNK):
            pltpu.sync_copy(rows0, acc_sh.at[pl.ds(row0 + j * CHUNK, CHUNK)])

    plsc.subcore_barrier()

    rows = (rows0, rows1)
    gsem = (gsem0, gsem1)

    def gather(j, b):
        pltpu.async_copy(h2_sh.at[src_all.at[j]], rows[b], gsem[b])

    def gwait(b):
        pltpu.make_async_copy(h2_sh.at[src_all.at[0]], rows[b], gsem[b]).wait()

    def scatter(j, b):
        pltpu.sync_copy(rows[b], acc_sh.at[dst_all.at[j]], add=True)

    gather(0, 0)

    def body(i, carry):
        j = 2 * i
        gather(j + 1, 1)
        gwait(0)
        scatter(j, 0)
        gather(j + 2, 0)
        gwait(1)
        scatter(j + 1, 1)
        return carry

    lax.fori_loop(0, (CPT - 2) // 2, body, 0)
    gather(CPT - 1, 1)
    gwait(0)
    scatter(CPT - 2, 0)
    gwait(1)
    scatter(CPT - 1, 1)

    plsc.subcore_barrier()
    pltpu.sync_copy(acc_sh.at[pl.ds(row0, RPT)],
                    accp_hbm.at[c, pl.ds(row0, RPT)])


def _mm_tc_body(x_ref, w_ref, g_ref):
    g_ref[...] = jnp.dot(x_ref[...], w_ref[...],
                         preferred_element_type=jnp.float32)


def _scale_tc_body(g_ref, deg0_ref, deg1_ref, h2_ref, dis_ref):
    deg = deg0_ref[...] + deg1_ref[...] + 1.0
    dis = lax.rsqrt(deg)
    h2_ref[...] = g_ref[...] * dis[:, None]
    dis_ref[...] = dis


def _final_tc_body(a0_ref, a1_ref, disr_ref, b_ref, out_ref):
    a = a0_ref[0] + a1_ref[0]
    out_ref[...] = a * disr_ref[...] + b_ref[...]


def kernel(x, edge_index, W, b):
    pad = E_PAD - N_EDGES
    ei3 = jnp.pad(edge_index.astype(jnp.int32), ((0, 0), (0, pad)),
                  constant_values=N_NODES).reshape(2, NW, CPT, CHUNK)
    b2 = jnp.concatenate([b, b]).reshape(1, 2 * D_OUT)

    mesh = plsc.VectorSubcoreMesh(core_axis_name="c", subcore_axis_name="s")

    deg_k = functools.partial(
        pl.kernel,
        out_type=jax.ShapeDtypeStruct((NC * N_PAD,), jnp.float32),
        mesh=mesh,
        scratch_types=[
            pltpu.VMEM((CPT, CHUNK), jnp.int32),
            pltpu.VMEM((CHUNK,), jnp.float32),
            pltpu.VMEM((RPT,), jnp.float32),
            pltpu.VMEM_SHARED((N_PAD,), jnp.float32),
            pltpu.SemaphoreType.DMA,
        ],
    )(_deg_body)
    degp = deg_k(ei3)

    MM_BLK = 2048
    mm_blocks = N_PAD // MM_BLK
    g = pl.pallas_call(
        _mm_tc_body,
        grid=(mm_blocks,),
        in_specs=[
            pl.BlockSpec((MM_BLK, D_IN), lambda i: (i, 0)),
            pl.BlockSpec((D_IN, D_OUT), lambda i: (0, 0)),
        ],
        out_specs=pl.BlockSpec((MM_BLK, D_OUT), lambda i: (i, 0)),
        out_shape=jax.ShapeDtypeStruct((N_PAD, D_OUT), jnp.float32),
    )(x, W)

    h2, dis = pl.pallas_call(
        _scale_tc_body,
        grid=(mm_blocks,),
        in_specs=[
            pl.BlockSpec((MM_BLK, D_OUT), lambda i: (i, 0)),
            pl.BlockSpec((MM_BLK,), lambda i: (i,)),
            pl.BlockSpec((MM_BLK,), lambda i: (i + mm_blocks,)),
        ],
        out_specs=[
            pl.BlockSpec((MM_BLK, D_OUT), lambda i: (i, 0)),
            pl.BlockSpec((MM_BLK,), lambda i: (i,)),
        ],
        out_shape=[
            jax.ShapeDtypeStruct((N_PAD, D_OUT), jnp.float32),
            jax.ShapeDtypeStruct((N_PAD,), jnp.float32),
        ],
    )(g, degp, degp)
    disr = jnp.broadcast_to(
        dis.reshape(N_PAD // 2, 2, 1), (N_PAD // 2, 2, D_OUT)
    ).reshape(N_PAD // 2, 2 * D_OUT)

    scat_k = functools.partial(
        pl.kernel,
        out_type=jax.ShapeDtypeStruct((NC, N_PAD, D_OUT), jnp.float32),
        mesh=mesh,
        compiler_params=pltpu.CompilerParams(use_tc_tiling_on_sc=False),
        scratch_types=[
            pltpu.VMEM((CPT, CHUNK), jnp.int32),
            pltpu.VMEM((CPT, CHUNK), jnp.int32),
            pltpu.VMEM((CHUNK, D_OUT), jnp.float32),
            pltpu.VMEM((CHUNK, D_OUT), jnp.float32),
            pltpu.VMEM_SHARED((N_PAD, D_OUT), jnp.float32),
            pltpu.VMEM_SHARED((N_PAD, D_OUT), jnp.float32),
            pltpu.SemaphoreType.DMA,
            pltpu.SemaphoreType.DMA,
        ],
    )(_scatter_body)
    accp = scat_k(ei3, h2)
    ap = accp.reshape(NC, N_PAD // 2, 2 * D_OUT)

    out_blocks = -(-(N_NODES // 2) // (ROW_BLK // 2))
    outp = pl.pallas_call(
        _final_tc_body,
        grid=(out_blocks,),
        in_specs=[
            pl.BlockSpec((1, ROW_BLK // 2, 2 * D_OUT), lambda i: (0, i, 0)),
            pl.BlockSpec((1, ROW_BLK // 2, 2 * D_OUT), lambda i: (1, i, 0)),
            pl.BlockSpec((ROW_BLK // 2, 2 * D_OUT), lambda i: (i, 0)),
            pl.BlockSpec((1, 2 * D_OUT), lambda i: (0, 0)),
        ],
        out_specs=pl.BlockSpec((ROW_BLK // 2, 2 * D_OUT), lambda i: (i, 0)),
        out_shape=jax.ShapeDtypeStruct((N_NODES // 2, 2 * D_OUT), jnp.float32),
    )(ap, ap, disr, b2)

    return outp.reshape(N_NODES, D_OUT)


# speedup vs baseline: 2.8702x; 1.0910x over previous
import functools

import jax
import jax.numpy as jnp
from jax import lax
from jax.experimental import pallas as pl
from jax.experimental.pallas import tpu as pltpu
from jax.experimental.pallas import tpu_sc as plsc

N_NODES = 10000
N_EDGES = 320000
D_IN = 128
D_OUT = 64

NC, NS, L = 2, 16, 16
NW = NC * NS
CHUNK = 128
CPT = 2 * (-(-N_EDGES // (NW * CHUNK * 2)))
E_PAD = NW * CPT * CHUNK
ROW_BLK = 512
N_PAD = ROW_BLK * (-(-(N_NODES + 1) // ROW_BLK))
RPT = N_PAD // NS


def _deg_body(ei_hbm, degp_hbm, dst_all, ones_v, zcol_v, deg_sh, ssem):
    c = lax.axis_index("c")
    s = lax.axis_index("s")
    wid = c * NS + s
    row0 = s * RPT

    pltpu.sync_copy(ei_hbm.at[1, wid], dst_all)
    for i in range(CHUNK // L):
        ones_v[pl.ds(i * L, L)] = jnp.ones((L,), jnp.float32)

    def zero(i, carry):
        zcol_v[pl.ds(i * L, L)] = jnp.zeros((L,), jnp.float32)
        return carry

    lax.fori_loop(0, RPT // L, zero, 0)
    pltpu.sync_copy(zcol_v, deg_sh.at[pl.ds(row0, RPT)])
    plsc.subcore_barrier()

    def body(j, carry):
        pltpu.async_copy(ones_v, deg_sh.at[dst_all.at[j]], ssem, add=True)
        return carry

    lax.fori_loop(0, CPT, body, 0)

    def drain(j, carry):
        pltpu.make_async_copy(ones_v, deg_sh.at[dst_all.at[0]], ssem).wait()
        return carry

    lax.fori_loop(0, CPT, drain, 0)
    plsc.subcore_barrier()
    pltpu.sync_copy(deg_sh.at[pl.ds(row0, RPT)],
                    degp_hbm.at[pl.ds(c * N_PAD + row0, RPT)])


def _scatter_body(ei_hbm, h2_hbm, accp_hbm,
                  src_all, dst_all, rows0, rows1, rows2,
                  h2_sh, acc_sh,
                  gsem0, gsem1, gsem2, ssem0, ssem1, ssem2):
    c = lax.axis_index("c")
    s = lax.axis_index("s")
    wid = c * NS + s
    row0 = s * RPT

    pltpu.sync_copy(ei_hbm.at[0, wid], src_all)
    pltpu.sync_copy(ei_hbm.at[1, wid], dst_all)
    pltpu.sync_copy(h2_hbm.at[pl.ds(row0, RPT)], h2_sh.at[pl.ds(row0, RPT)])

    @pl.when(c == 0)
    def _():
        pltpu.sync_copy(h2_hbm.at[pl.ds(row0, RPT)], acc_sh.at[pl.ds(row0, RPT)])

    @pl.when(c == 1)
    def _():
        def zero(i, carry):
            for k in range(D_OUT // L):
                rows0[i, pl.ds(k * L, L)] = jnp.zeros((L,), jnp.float32)
            return carry

        lax.fori_loop(0, CHUNK, zero, 0)
        for j in range(RPT // CHUNK):
            pltpu.sync_copy(rows0, acc_sh.at[pl.ds(row0 + j * CHUNK, CHUNK)])

    plsc.subcore_barrier()

    rows = (rows0, rows1, rows2)
    gsem = (gsem0, gsem1, gsem2)
    ssem = (ssem0, ssem1, ssem2)

    def gather(j, b):
        pltpu.async_copy(h2_sh.at[src_all.at[j]], rows[b], gsem[b])

    def gwait(b):
        pltpu.make_async_copy(h2_sh.at[src_all.at[0]], rows[b], gsem[b]).wait()

    def sstart(j, b):
        pltpu.async_copy(rows[b], acc_sh.at[dst_all.at[j]], ssem[b], add=True)

    def swait(b):
        pltpu.make_async_copy(rows[b], acc_sh.at[dst_all.at[0]],
                              ssem[b]).wait()

    gather(0, 0)
    gather(1, 1)
    gwait(0)
    sstart(0, 0)
    gather(2, 2)
    gwait(1)
    sstart(1, 1)

    def body(i, carry):
        for u in range(3):
            j3 = 3 * i + 2 + u
            b = (2 + u) % 3
            swait(u)
            gather(j3 + 1, u)
            gwait(b)
            sstart(j3, b)
        return carry

    lax.fori_loop(0, (CPT - 5) // 3, body, 0)
    for j in range(CPT - 3, CPT):
        swait((j + 1) % 3)
        if j + 1 < CPT:
            gather(j + 1, (j + 1) % 3)
        gwait(j % 3)
        sstart(j, j % 3)
    swait((CPT - 2) % 3)
    swait((CPT - 1) % 3)

    plsc.subcore_barrier()
    pltpu.sync_copy(acc_sh.at[pl.ds(row0, RPT)],
                    accp_hbm.at[c, pl.ds(row0, RPT)])


def _mm_tc_body(x_ref, w_ref, g_ref):
    g_ref[...] = jnp.dot(x_ref[...], w_ref[...],
                         preferred_element_type=jnp.float32)


def _scale_tc_body(g_ref, deg0_ref, deg1_ref, h2_ref, dis_ref):
    deg = deg0_ref[...] + deg1_ref[...] + 1.0
    dis = lax.rsqrt(deg)
    h2_ref[...] = g_ref[...] * dis[:, None]
    dis_ref[...] = dis


def _final_tc_body(a0_ref, a1_ref, disr_ref, b_ref, out_ref):
    a = a0_ref[0] + a1_ref[0]
    out_ref[...] = a * disr_ref[...] + b_ref[...]


def kernel(x, edge_index, W, b):
    pad = E_PAD - N_EDGES
    ei3 = jnp.pad(edge_index.astype(jnp.int32), ((0, 0), (0, pad)),
                  constant_values=N_NODES).reshape(2, NW, CPT, CHUNK)
    b2 = jnp.concatenate([b, b]).reshape(1, 2 * D_OUT)

    mesh = plsc.VectorSubcoreMesh(core_axis_name="c", subcore_axis_name="s")

    deg_k = functools.partial(
        pl.kernel,
        out_type=jax.ShapeDtypeStruct((NC * N_PAD,), jnp.float32),
        mesh=mesh,
        scratch_types=[
            pltpu.VMEM((CPT, CHUNK), jnp.int32),
            pltpu.VMEM((CHUNK,), jnp.float32),
            pltpu.VMEM((RPT,), jnp.float32),
            pltpu.VMEM_SHARED((N_PAD,), jnp.float32),
            pltpu.SemaphoreType.DMA,
        ],
    )(_deg_body)
    degp = deg_k(ei3)

    MM_BLK = 2048
    mm_blocks = N_PAD // MM_BLK
    g = pl.pallas_call(
        _mm_tc_body,
        grid=(mm_blocks,),
        in_specs=[
            pl.BlockSpec((MM_BLK, D_IN), lambda i: (i, 0)),
            pl.BlockSpec((D_IN, D_OUT), lambda i: (0, 0)),
        ],
        out_specs=pl.BlockSpec((MM_BLK, D_OUT), lambda i: (i, 0)),
        out_shape=jax.ShapeDtypeStruct((N_PAD, D_OUT), jnp.float32),
    )(x, W)

    h2, dis = pl.pallas_call(
        _scale_tc_body,
        grid=(mm_blocks,),
        in_specs=[
            pl.BlockSpec((MM_BLK, D_OUT), lambda i: (i, 0)),
            pl.BlockSpec((MM_BLK,), lambda i: (i,)),
            pl.BlockSpec((MM_BLK,), lambda i: (i + mm_blocks,)),
        ],
        out_specs=[
            pl.BlockSpec((MM_BLK, D_OUT), lambda i: (i, 0)),
            pl.BlockSpec((MM_BLK,), lambda i: (i,)),
        ],
        out_shape=[
            jax.ShapeDtypeStruct((N_PAD, D_OUT), jnp.float32),
            jax.ShapeDtypeStruct((N_PAD,), jnp.float32),
        ],
    )(g, degp, degp)
    disr = jnp.broadcast_to(
        dis.reshape(N_PAD // 2, 2, 1), (N_PAD // 2, 2, D_OUT)
    ).reshape(N_PAD // 2, 2 * D_OUT)

    scat_k = functools.partial(
        pl.kernel,
        out_type=jax.ShapeDtypeStruct((NC, N_PAD, D_OUT), jnp.float32),
        mesh=mesh,
        compiler_params=pltpu.CompilerParams(use_tc_tiling_on_sc=False),
        scratch_types=(
            [
                pltpu.VMEM((CPT, CHUNK), jnp.int32),
                pltpu.VMEM((CPT, CHUNK), jnp.int32),
            ]
            + [pltpu.VMEM((CHUNK, D_OUT), jnp.float32)] * 3
            + [
                pltpu.VMEM_SHARED((N_PAD, D_OUT), jnp.float32),
                pltpu.VMEM_SHARED((N_PAD, D_OUT), jnp.float32),
            ]
            + [pltpu.SemaphoreType.DMA] * 6
        ),
    )(_scatter_body)
    accp = scat_k(ei3, h2)
    ap = accp.reshape(NC, N_PAD // 2, 2 * D_OUT)

    out_blocks = -(-(N_NODES // 2) // (ROW_BLK // 2))
    outp = pl.pallas_call(
        _final_tc_body,
        grid=(out_blocks,),
        in_specs=[
            pl.BlockSpec((1, ROW_BLK // 2, 2 * D_OUT), lambda i: (0, i, 0)),
            pl.BlockSpec((1, ROW_BLK // 2, 2 * D_OUT), lambda i: (1, i, 0)),
            pl.BlockSpec((ROW_BLK // 2, 2 * D_OUT), lambda i: (i, 0)),
            pl.BlockSpec((1, 2 * D_OUT), lambda i: (0, 0)),
        ],
        out_specs=pl.BlockSpec((ROW_BLK // 2, 2 * D_OUT), lambda i: (i, 0)),
        out_shape=jax.ShapeDtypeStruct((N_NODES // 2, 2 * D_OUT), jnp.float32),
    )(ap, ap, disr, b2)

    return outp.reshape(N_NODES, D_OUT)


# speedup vs baseline: 2.8956x; 1.0089x over previous
import functools

import jax
import jax.numpy as jnp
from jax import lax
from jax.experimental import pallas as pl
from jax.experimental.pallas import tpu as pltpu
from jax.experimental.pallas import tpu_sc as plsc

N_NODES = 10000
N_EDGES = 320000
D_IN = 128
D_OUT = 64

NC, NS, L = 2, 16, 16
NW = NC * NS
CHUNK = 128
CPT = 2 * (-(-N_EDGES // (NW * CHUNK * 2)))
E_PAD = NW * CPT * CHUNK
ROW_BLK = 512
N_PAD = ROW_BLK * (-(-(N_NODES + 1) // ROW_BLK))
RPT = N_PAD // NS


DEG_K0 = 104
DEG_K1 = NW * CPT // NS - DEG_K0


def _deg_body(ei_hbm, degp_hbm, dst_all, ones_v, zcol_v, deg_sh, ssem):
    c = lax.axis_index("c")
    s = lax.axis_index("s")
    row0 = s * RPT

    for i in range(CHUNK // L):
        ones_v[pl.ds(i * L, L)] = jnp.ones((L,), jnp.float32)

    def zero(i, carry):
        zcol_v[pl.ds(i * L, L)] = jnp.zeros((L,), jnp.float32)
        return carry

    lax.fori_loop(0, RPT // L, zero, 0)
    pltpu.sync_copy(zcol_v, deg_sh.at[pl.ds(row0, RPT)])
    plsc.subcore_barrier()

    def run(base, k):
        pltpu.sync_copy(ei_hbm.at[1, pl.ds(base, k)], dst_all.at[pl.ds(0, k)])

        def body(j, carry):
            pltpu.async_copy(ones_v, deg_sh.at[dst_all.at[j]], ssem, add=True)
            return carry

        lax.fori_loop(0, k, body, 0)

        def drain(j, carry):
            pltpu.make_async_copy(ones_v, deg_sh.at[dst_all.at[0]],
                                  ssem).wait()
            return carry

        lax.fori_loop(0, k, drain, 0)

    @pl.when(c == 0)
    def _():
        run(s * DEG_K0, DEG_K0)

    @pl.when(c == 1)
    def _():
        run(NS * DEG_K0 + s * DEG_K1, DEG_K1)

    plsc.subcore_barrier()
    pltpu.sync_copy(deg_sh.at[pl.ds(row0, RPT)],
                    degp_hbm.at[pl.ds(c * N_PAD + row0, RPT)])


def _scatter_body(ei_hbm, h2_hbm, accp_hbm,
                  src_all, dst_all, rows0, rows1, rows2,
                  h2_sh, acc_sh,
                  gsem0, gsem1, gsem2, ssem0, ssem1, ssem2):
    c = lax.axis_index("c")
    s = lax.axis_index("s")
    wid = c * NS + s
    row0 = s * RPT

    pltpu.sync_copy(ei_hbm.at[0, wid], src_all)
    pltpu.sync_copy(ei_hbm.at[1, wid], dst_all)
    pltpu.sync_copy(h2_hbm.at[pl.ds(row0, RPT)], h2_sh.at[pl.ds(row0, RPT)])

    @pl.when(c == 0)
    def _():
        pltpu.sync_copy(h2_hbm.at[pl.ds(row0, RPT)], acc_sh.at[pl.ds(row0, RPT)])

    @pl.when(c == 1)
    def _():
        def zero(i, carry):
            for k in range(D_OUT // L):
                rows0[i, pl.ds(k * L, L)] = jnp.zeros((L,), jnp.float32)
            return carry

        lax.fori_loop(0, CHUNK, zero, 0)
        for j in range(RPT // CHUNK):
            pltpu.sync_copy(rows0, acc_sh.at[pl.ds(row0 + j * CHUNK, CHUNK)])

    plsc.subcore_barrier()

    rows = (rows0, rows1, rows2)
    gsem = (gsem0, gsem1, gsem2)
    ssem = (ssem0, ssem1, ssem2)

    def gather(j, b):
        pltpu.async_copy(h2_sh.at[src_all.at[j]], rows[b], gsem[b])

    def gwait(b):
        pltpu.make_async_copy(h2_sh.at[src_all.at[0]], rows[b], gsem[b]).wait()

    def sstart(j, b):
        pltpu.async_copy(rows[b], acc_sh.at[dst_all.at[j]], ssem[b], add=True)

    def swait(b):
        pltpu.make_async_copy(rows[b], acc_sh.at[dst_all.at[0]],
                              ssem[b]).wait()

    gather(0, 0)
    gather(1, 1)
    gwait(0)
    sstart(0, 0)
    gather(2, 2)
    gwait(1)
    sstart(1, 1)

    def body(i, carry):
        for u in range(3):
            j3 = 3 * i + 2 + u
            b = (2 + u) % 3
            swait(u)
            gather(j3 + 1, u)
            gwait(b)
            sstart(j3, b)
        return carry

    lax.fori_loop(0, (CPT - 5) // 3, body, 0)
    for j in range(CPT - 3, CPT):
        swait((j + 1) % 3)
        if j + 1 < CPT:
            gather(j + 1, (j + 1) % 3)
        gwait(j % 3)
        sstart(j, j % 3)
    swait((CPT - 2) % 3)
    swait((CPT - 1) % 3)

    plsc.subcore_barrier()
    pltpu.sync_copy(acc_sh.at[pl.ds(row0, RPT)],
                    accp_hbm.at[c, pl.ds(row0, RPT)])


def _mm_tc_body(x_ref, w_ref, g_ref):
    g_ref[...] = jnp.dot(x_ref[...], w_ref[...],
                         preferred_element_type=jnp.float32)


def _scale_tc_body(g_ref, deg0_ref, deg1_ref, h2_ref, dis_ref):
    deg = deg0_ref[...] + deg1_ref[...] + 1.0
    dis = lax.rsqrt(deg)
    h2_ref[...] = g_ref[...] * dis[:, None]
    dis_ref[...] = dis


def _final_tc_body(a0_ref, a1_ref, disr_ref, b_ref, out_ref):
    a = a0_ref[0] + a1_ref[0]
    out_ref[...] = a * disr_ref[...] + b_ref[...]


def kernel(x, edge_index, W, b):
    pad = E_PAD - N_EDGES
    ei3 = jnp.pad(edge_index.astype(jnp.int32), ((0, 0), (0, pad)),
                  constant_values=N_NODES).reshape(2, NW, CPT, CHUNK)
    b2 = jnp.concatenate([b, b]).reshape(1, 2 * D_OUT)

    mesh = plsc.VectorSubcoreMesh(core_axis_name="c", subcore_axis_name="s")

    deg_k = functools.partial(
        pl.kernel,
        out_type=jax.ShapeDtypeStruct((NC * N_PAD,), jnp.float32),
        mesh=mesh,
        scratch_types=[
            pltpu.VMEM((DEG_K0, CHUNK), jnp.int32),
            pltpu.VMEM((CHUNK,), jnp.float32),
            pltpu.VMEM((RPT,), jnp.float32),
            pltpu.VMEM_SHARED((N_PAD,), jnp.float32),
            pltpu.SemaphoreType.DMA,
        ],
    )(_deg_body)
    degp = deg_k(ei3.reshape(2, NW * CPT, CHUNK))

    MM_BLK = 2048
    mm_blocks = N_PAD // MM_BLK
    g = pl.pallas_call(
        _mm_tc_body,
        grid=(mm_blocks,),
        in_specs=[
            pl.BlockSpec((MM_BLK, D_IN), lambda i: (i, 0)),
            pl.BlockSpec((D_IN, D_OUT), lambda i: (0, 0)),
        ],
        out_specs=pl.BlockSpec((MM_BLK, D_OUT), lambda i: (i, 0)),
        out_shape=jax.ShapeDtypeStruct((N_PAD, D_OUT), jnp.float32),
    )(x, W)

    h2, dis = pl.pallas_call(
        _scale_tc_body,
        grid=(mm_blocks,),
        in_specs=[
            pl.BlockSpec((MM_BLK, D_OUT), lambda i: (i, 0)),
            pl.BlockSpec((MM_BLK,), lambda i: (i,)),
            pl.BlockSpec((MM_BLK,), lambda i: (i + mm_blocks,)),
        ],
        out_specs=[
            pl.BlockSpec((MM_BLK, D_OUT), lambda i: (i, 0)),
            pl.BlockSpec((MM_BLK,), lambda i: (i,)),
        ],
        out_shape=[
            jax.ShapeDtypeStruct((N_PAD, D_OUT), jnp.float32),
            jax.ShapeDtypeStruct((N_PAD,), jnp.float32),
        ],
    )(g, degp, degp)
    disr = jnp.broadcast_to(
        dis.reshape(N_PAD // 2, 2, 1), (N_PAD // 2, 2, D_OUT)
    ).reshape(N_PAD // 2, 2 * D_OUT)

    scat_k = functools.partial(
        pl.kernel,
        out_type=jax.ShapeDtypeStruct((NC, N_PAD, D_OUT), jnp.float32),
        mesh=mesh,
        compiler_params=pltpu.CompilerParams(use_tc_tiling_on_sc=False),
        scratch_types=(
            [
                pltpu.VMEM((CPT, CHUNK), jnp.int32),
                pltpu.VMEM((CPT, CHUNK), jnp.int32),
            ]
            + [pltpu.VMEM((CHUNK, D_OUT), jnp.float32)] * 3
            + [
                pltpu.VMEM_SHARED((N_PAD, D_OUT), jnp.float32),
                pltpu.VMEM_SHARED((N_PAD, D_OUT), jnp.float32),
            ]
            + [pltpu.SemaphoreType.DMA] * 6
        ),
    )(_scatter_body)
    accp = scat_k(ei3, h2)
    ap = accp.reshape(NC, N_PAD // 2, 2 * D_OUT)

    out_blocks = -(-(N_NODES // 2) // (ROW_BLK // 2))
    outp = pl.pallas_call(
        _final_tc_body,
        grid=(out_blocks,),
        in_specs=[
            pl.BlockSpec((1, ROW_BLK // 2, 2 * D_OUT), lambda i: (0, i, 0)),
            pl.BlockSpec((1, ROW_BLK // 2, 2 * D_OUT), lambda i: (1, i, 0)),
            pl.BlockSpec((ROW_BLK // 2, 2 * D_OUT), lambda i: (i, 0)),
            pl.BlockSpec((1, 2 * D_OUT), lambda i: (0, 0)),
        ],
        out_specs=pl.BlockSpec((ROW_BLK // 2, 2 * D_OUT), lambda i: (i, 0)),
        out_shape=jax.ShapeDtypeStruct((N_NODES // 2, 2 * D_OUT), jnp.float32),
    )(ap, ap, disr, b2)

    return outp.reshape(N_NODES, D_OUT)


# speedup vs baseline: 2.9095x; 1.0048x over previous
import functools

import jax
import jax.numpy as jnp
from jax import lax
from jax.experimental import pallas as pl
from jax.experimental.pallas import tpu as pltpu
from jax.experimental.pallas import tpu_sc as plsc

N_NODES = 10000
N_EDGES = 320000
D_IN = 128
D_OUT = 64

NC, NS, L = 2, 16, 16
NW = NC * NS
CHUNK = 128
CPT = 2 * (-(-N_EDGES // (NW * CHUNK * 2)))
E_PAD = NW * CPT * CHUNK
ROW_BLK = 512
N_PAD = ROW_BLK * (-(-(N_NODES + 1) // ROW_BLK))
RPT = N_PAD // NS


DEG_K0 = 104
DEG_K1 = NW * CPT // NS - DEG_K0


def _deg_body(ei_hbm, degp_hbm, dst_all, ones_v, zcol_v, deg_sh, ssem):
    c = lax.axis_index("c")
    s = lax.axis_index("s")
    row0 = s * RPT

    for i in range(CHUNK // L):
        ones_v[pl.ds(i * L, L)] = jnp.ones((L,), jnp.float32)

    def zero(i, carry):
        zcol_v[pl.ds(i * L, L)] = jnp.zeros((L,), jnp.float32)
        return carry

    lax.fori_loop(0, RPT // L, zero, 0)
    pltpu.sync_copy(zcol_v, deg_sh.at[pl.ds(row0, RPT)])
    plsc.subcore_barrier()

    def run(base, k):
        pltpu.sync_copy(ei_hbm.at[1, pl.ds(base, k)], dst_all.at[pl.ds(0, k)])

        def body(j, carry):
            pltpu.async_copy(ones_v, deg_sh.at[dst_all.at[j]], ssem, add=True)
            return carry

        lax.fori_loop(0, k, body, 0)

        def drain(j, carry):
            pltpu.make_async_copy(ones_v, deg_sh.at[dst_all.at[0]],
                                  ssem).wait()
            return carry

        lax.fori_loop(0, k, drain, 0)

    @pl.when(c == 0)
    def _():
        run(s * DEG_K0, DEG_K0)

    @pl.when(c == 1)
    def _():
        run(NS * DEG_K0 + s * DEG_K1, DEG_K1)

    plsc.subcore_barrier()
    pltpu.sync_copy(deg_sh.at[pl.ds(row0, RPT)],
                    degp_hbm.at[pl.ds(c * N_PAD + row0, RPT)])


SCAT_K0 = 83
SCAT_K1 = NW * CPT // NS - SCAT_K0


def _scatter_body(ei_hbm, h2_hbm, accp_hbm,
                  src_all, dst_all, rows0, rows1, rows2,
                  h2_sh, acc_sh,
                  gsem0, gsem1, gsem2, ssem0, ssem1, ssem2):
    c = lax.axis_index("c")
    s = lax.axis_index("s")
    row0 = s * RPT

    pltpu.sync_copy(h2_hbm.at[pl.ds(row0, RPT)], h2_sh.at[pl.ds(row0, RPT)])

    @pl.when(c == 0)
    def _():
        pltpu.sync_copy(h2_hbm.at[pl.ds(row0, RPT)], acc_sh.at[pl.ds(row0, RPT)])

    @pl.when(c == 1)
    def _():
        def zero(i, carry):
            for k in range(D_OUT // L):
                rows0[i, pl.ds(k * L, L)] = jnp.zeros((L,), jnp.float32)
            return carry

        lax.fori_loop(0, CHUNK, zero, 0)
        for j in range(RPT // CHUNK):
            pltpu.sync_copy(rows0, acc_sh.at[pl.ds(row0 + j * CHUNK, CHUNK)])

    plsc.subcore_barrier()

    rows = (rows0, rows1, rows2)
    gsem = (gsem0, gsem1, gsem2)
    ssem = (ssem0, ssem1, ssem2)

    def pipeline(base, k, srcs):
        pltpu.sync_copy(ei_hbm.at[0, pl.ds(base, k)], src_all.at[pl.ds(0, k)])
        pltpu.sync_copy(ei_hbm.at[1, pl.ds(base, k)], dst_all.at[pl.ds(0, k)])

        def gather(j, b):
            pltpu.async_copy(srcs[b].at[src_all.at[j]], rows[b], gsem[b])

        def gwait(b):
            pltpu.make_async_copy(srcs[b].at[src_all.at[0]], rows[b],
                                  gsem[b]).wait()

        def sstart(j, b):
            pltpu.async_copy(rows[b], acc_sh.at[dst_all.at[j]], ssem[b],
                             add=True)

        def swait(b):
            pltpu.make_async_copy(rows[b], acc_sh.at[dst_all.at[0]],
                                  ssem[b]).wait()

        gather(0, 0)
        gather(1, 1)
        gwait(0)
        sstart(0, 0)
        gather(2, 2)
        gwait(1)
        sstart(1, 1)

        def body(i, carry):
            for u in range(3):
                j3 = 3 * i + 2 + u
                b = (2 + u) % 3
                swait(u)
                gather(j3 + 1, u)
                gwait(b)
                sstart(j3, b)
            return carry

        lax.fori_loop(0, (k - 5) // 3, body, 0)
        for j in range(k - 3, k):
            swait((j + 1) % 3)
            if j + 1 < k:
                gather(j + 1, (j + 1) % 3)
            gwait(j % 3)
            sstart(j, j % 3)
        swait((k - 2) % 3)
        swait((k - 1) % 3)

    @pl.when(c == 0)
    def _():
        pipeline(s * SCAT_K0, SCAT_K0, (h2_sh, h2_sh, h2_sh))

    @pl.when(c == 1)
    def _():
        pipeline(NS * SCAT_K0 + s * SCAT_K1, SCAT_K1, (h2_sh, h2_sh, h2_sh))

    plsc.subcore_barrier()
    pltpu.sync_copy(acc_sh.at[pl.ds(row0, RPT)],
                    accp_hbm.at[c, pl.ds(row0, RPT)])


def _mm_tc_body(x_ref, w_ref, g_ref):
    g_ref[...] = jnp.dot(x_ref[...], w_ref[...],
                         preferred_element_type=jnp.float32)


def _scale_tc_body(g_ref, deg0_ref, deg1_ref, h2_ref, dis_ref):
    deg = deg0_ref[...] + deg1_ref[...] + 1.0
    dis = lax.rsqrt(deg)
    h2_ref[...] = g_ref[...] * dis[:, None]
    dis_ref[...] = dis


def _final_tc_body(a0_ref, a1_ref, disr_ref, b_ref, out_ref):
    a = a0_ref[0] + a1_ref[0]
    out_ref[...] = a * disr_ref[...] + b_ref[...]


def kernel(x, edge_index, W, b):
    pad = E_PAD - N_EDGES
    ei3 = jnp.pad(edge_index.astype(jnp.int32), ((0, 0), (0, pad)),
                  constant_values=N_NODES).reshape(2, NW, CPT, CHUNK)
    b2 = jnp.concatenate([b, b]).reshape(1, 2 * D_OUT)

    mesh = plsc.VectorSubcoreMesh(core_axis_name="c", subcore_axis_name="s")

    deg_k = functools.partial(
        pl.kernel,
        out_type=jax.ShapeDtypeStruct((NC * N_PAD,), jnp.float32),
        mesh=mesh,
        scratch_types=[
            pltpu.VMEM((DEG_K0, CHUNK), jnp.int32),
            pltpu.VMEM((CHUNK,), jnp.float32),
            pltpu.VMEM((RPT,), jnp.float32),
            pltpu.VMEM_SHARED((N_PAD,), jnp.float32),
            pltpu.SemaphoreType.DMA,
        ],
    )(_deg_body)
    degp = deg_k(ei3.reshape(2, NW * CPT, CHUNK))

    MM_BLK = 2048
    mm_blocks = N_PAD // MM_BLK
    g = pl.pallas_call(
        _mm_tc_body,
        grid=(mm_blocks,),
        in_specs=[
            pl.BlockSpec((MM_BLK, D_IN), lambda i: (i, 0)),
            pl.BlockSpec((D_IN, D_OUT), lambda i: (0, 0)),
        ],
        out_specs=pl.BlockSpec((MM_BLK, D_OUT), lambda i: (i, 0)),
        out_shape=jax.ShapeDtypeStruct((N_PAD, D_OUT), jnp.float32),
    )(x, W)

    h2, dis = pl.pallas_call(
        _scale_tc_body,
        grid=(mm_blocks,),
        in_specs=[
            pl.BlockSpec((MM_BLK, D_OUT), lambda i: (i, 0)),
            pl.BlockSpec((MM_BLK,), lambda i: (i,)),
            pl.BlockSpec((MM_BLK,), lambda i: (i + mm_blocks,)),
        ],
        out_specs=[
            pl.BlockSpec((MM_BLK, D_OUT), lambda i: (i, 0)),
            pl.BlockSpec((MM_BLK,), lambda i: (i,)),
        ],
        out_shape=[
            jax.ShapeDtypeStruct((N_PAD, D_OUT), jnp.float32),
            jax.ShapeDtypeStruct((N_PAD,), jnp.float32),
        ],
    )(g, degp, degp)
    disr = jnp.broadcast_to(
        dis.reshape(N_PAD // 2, 2, 1), (N_PAD // 2, 2, D_OUT)
    ).reshape(N_PAD // 2, 2 * D_OUT)

    scat_k = functools.partial(
        pl.kernel,
        out_type=jax.ShapeDtypeStruct((NC, N_PAD, D_OUT), jnp.float32),
        mesh=mesh,
        compiler_params=pltpu.CompilerParams(use_tc_tiling_on_sc=False),
        scratch_types=(
            [
                pltpu.VMEM((SCAT_K0, CHUNK), jnp.int32),
                pltpu.VMEM((SCAT_K0, CHUNK), jnp.int32),
            ]
            + [pltpu.VMEM((CHUNK, D_OUT), jnp.float32)] * 3
            + [
                pltpu.VMEM_SHARED((N_PAD, D_OUT), jnp.float32),
                pltpu.VMEM_SHARED((N_PAD, D_OUT), jnp.float32),
            ]
            + [pltpu.SemaphoreType.DMA] * 6
        ),
    )(_scatter_body)
    accp = scat_k(ei3.reshape(2, NW * CPT, CHUNK), h2)
    ap = accp.reshape(NC, N_PAD // 2, 2 * D_OUT)

    out_blocks = -(-(N_NODES // 2) // (ROW_BLK // 2))
    outp = pl.pallas_call(
        _final_tc_body,
        grid=(out_blocks,),
        in_specs=[
            pl.BlockSpec((1, ROW_BLK // 2, 2 * D_OUT), lambda i: (0, i, 0)),
            pl.BlockSpec((1, ROW_BLK // 2, 2 * D_OUT), lambda i: (1, i, 0)),
            pl.BlockSpec((ROW_BLK // 2, 2 * D_OUT), lambda i: (i, 0)),
            pl.BlockSpec((1, 2 * D_OUT), lambda i: (0, 0)),
        ],
        out_specs=pl.BlockSpec((ROW_BLK // 2, 2 * D_OUT), lambda i: (i, 0)),
        out_shape=jax.ShapeDtypeStruct((N_NODES // 2, 2 * D_OUT), jnp.float32),
    )(ap, ap, disr, b2)

    return outp.reshape(N_NODES, D_OUT)


# speedup vs baseline: 3.0219x; 1.0386x over previous
import functools

import jax
import jax.numpy as jnp
from jax import lax
from jax.experimental import pallas as pl
from jax.experimental.pallas import tpu as pltpu
from jax.experimental.pallas import tpu_sc as plsc

N_NODES = 10000
N_EDGES = 320000
D_IN = 128
D_OUT = 64

NC, NS, L = 2, 16, 16
NW = NC * NS
CHUNK = 128
CPT = 2 * (-(-N_EDGES // (NW * CHUNK * 2)))
E_PAD = NW * CPT * CHUNK
ROW_BLK = 512
N_PAD = ROW_BLK * (-(-(N_NODES + 1) // ROW_BLK))
RPT = N_PAD // NS


DEG_K0 = 104
DEG_K1 = NW * CPT // NS - DEG_K0


def _deg_body(ei_hbm, degp_hbm, dst_all, ones_v, zcol_v, deg_sh, ssem):
    c = lax.axis_index("c")
    s = lax.axis_index("s")
    row0 = s * RPT

    for i in range(CHUNK // L):
        ones_v[pl.ds(i * L, L)] = jnp.ones((L,), jnp.float32)

    def zero(i, carry):
        zcol_v[pl.ds(i * L, L)] = jnp.zeros((L,), jnp.float32)
        return carry

    lax.fori_loop(0, RPT // L, zero, 0)
    pltpu.sync_copy(zcol_v, deg_sh.at[pl.ds(row0, RPT)])
    plsc.subcore_barrier()

    def run(base, k):
        pltpu.sync_copy(ei_hbm.at[1, pl.ds(base, k)], dst_all.at[pl.ds(0, k)])

        def body(j, carry):
            pltpu.async_copy(ones_v, deg_sh.at[dst_all.at[j]], ssem, add=True)
            return carry

        lax.fori_loop(0, k, body, 0)

        def drain(j, carry):
            pltpu.make_async_copy(ones_v, deg_sh.at[dst_all.at[0]],
                                  ssem).wait()
            return carry

        lax.fori_loop(0, k, drain, 0)

    @pl.when(c == 0)
    def _():
        run(s * DEG_K0, DEG_K0)

    @pl.when(c == 1)
    def _():
        run(NS * DEG_K0 + s * DEG_K1, DEG_K1)

    plsc.subcore_barrier()
    pltpu.sync_copy(deg_sh.at[pl.ds(row0, RPT)],
                    degp_hbm.at[pl.ds(c * N_PAD + row0, RPT)])


SCAT_K0 = 83
SCAT_K1 = NW * CPT // NS - SCAT_K0


def _scatter_body(ei_hbm, h2_hbm, accp_hbm,
                  src_all, dst_all, rows0, rows1, rows2,
                  h2_sh, acc_sh,
                  gsem0, gsem1, gsem2, ssem0, ssem1, ssem2):
    c = lax.axis_index("c")
    s = lax.axis_index("s")
    row0 = s * RPT

    pltpu.sync_copy(h2_hbm.at[pl.ds(row0, RPT)], h2_sh.at[pl.ds(row0, RPT)])

    @pl.when(c == 0)
    def _():
        pltpu.sync_copy(h2_hbm.at[pl.ds(row0, RPT)], acc_sh.at[pl.ds(row0, RPT)])

    @pl.when(c == 1)
    def _():
        def zero(i, carry):
            for k in range(D_OUT // L):
                rows0[i, pl.ds(k * L, L)] = jnp.zeros((L,), jnp.float32)
            return carry

        lax.fori_loop(0, CHUNK, zero, 0)
        for j in range(RPT // CHUNK):
            pltpu.sync_copy(rows0, acc_sh.at[pl.ds(row0 + j * CHUNK, CHUNK)])

    plsc.subcore_barrier()

    rows = (rows0, rows1, rows2)
    gsem = (gsem0, gsem1, gsem2)
    ssem = (ssem0, ssem1, ssem2)

    def pipeline(base, k, srcs):
        pltpu.sync_copy(ei_hbm.at[0, pl.ds(base, k)], src_all.at[pl.ds(0, k)])
        pltpu.sync_copy(ei_hbm.at[1, pl.ds(base, k)], dst_all.at[pl.ds(0, k)])

        def gather(j, b):
            pltpu.async_copy(srcs[b].at[src_all.at[j]], rows[b], gsem[b])

        def gwait(b):
            pltpu.make_async_copy(srcs[b].at[src_all.at[0]], rows[b],
                                  gsem[b]).wait()

        def sstart(j, b):
            pltpu.async_copy(rows[b], acc_sh.at[dst_all.at[j]], ssem[b],
                             add=True)

        def swait(b):
            pltpu.make_async_copy(rows[b], acc_sh.at[dst_all.at[0]],
                                  ssem[b]).wait()

        gather(0, 0)
        gather(1, 1)
        gwait(0)
        sstart(0, 0)
        gather(2, 2)
        gwait(1)
        sstart(1, 1)

        def body(i, carry):
            for u in range(3):
                j3 = 3 * i + 2 + u
                b = (2 + u) % 3
                swait(u)
                gather(j3 + 1, u)
                gwait(b)
                sstart(j3, b)
            return carry

        lax.fori_loop(0, (k - 5) // 3, body, 0)
        for j in range(k - 3, k):
            swait((j + 1) % 3)
            if j + 1 < k:
                gather(j + 1, (j + 1) % 3)
            gwait(j % 3)
            sstart(j, j % 3)
        swait((k - 2) % 3)
        swait((k - 1) % 3)

    @pl.when(c == 0)
    def _():
        pipeline(s * SCAT_K0, SCAT_K0, (h2_sh, h2_sh, h2_sh))

    @pl.when(c == 1)
    def _():
        pipeline(NS * SCAT_K0 + s * SCAT_K1, SCAT_K1, (h2_sh, h2_sh, h2_sh))

    plsc.subcore_barrier()
    pltpu.sync_copy(acc_sh.at[pl.ds(row0, RPT)],
                    accp_hbm.at[c, pl.ds(row0, RPT)])


def _mm_tc_body(x_ref, w_ref, g_ref):
    g_ref[...] = jnp.dot(x_ref[...], w_ref[...],
                         preferred_element_type=jnp.float32)


def _scale_tc_body(g_ref, deg0_ref, deg1_ref, h2_ref, dis_ref):
    deg = deg0_ref[...] + deg1_ref[...] + 1.0
    dis = lax.rsqrt(deg)
    h2_ref[...] = g_ref[...] * dis[:, None]
    dis_ref[...] = dis


def _final_tc_body(a0_ref, a1_ref, disr_ref, b_ref, out_ref):
    a = a0_ref[0] + a1_ref[0]
    out_ref[...] = a * disr_ref[...] + b_ref[...]


def kernel(x, edge_index, W, b):
    pad = E_PAD - N_EDGES
    ei3 = jnp.pad(edge_index.astype(jnp.int32), ((0, 0), (0, pad)),
                  constant_values=N_NODES).reshape(2, NW, CPT, CHUNK)
    b2 = jnp.concatenate([b, b]).reshape(1, 2 * D_OUT)

    mesh = plsc.VectorSubcoreMesh(core_axis_name="c", subcore_axis_name="s")

    deg_k = functools.partial(
        pl.kernel,
        out_type=jax.ShapeDtypeStruct((NC * N_PAD,), jnp.float32),
        mesh=mesh,
        scratch_types=[
            pltpu.VMEM((DEG_K0, CHUNK), jnp.int32),
            pltpu.VMEM((CHUNK,), jnp.float32),
            pltpu.VMEM((RPT,), jnp.float32),
            pltpu.VMEM_SHARED((N_PAD,), jnp.float32),
            pltpu.SemaphoreType.DMA,
        ],
    )(_deg_body)
    degp = deg_k(ei3.reshape(2, NW * CPT, CHUNK))

    MM_BLK = 2048
    mm_blocks = N_PAD // MM_BLK
    g = pl.pallas_call(
        _mm_tc_body,
        grid=(mm_blocks,),
        in_specs=[
            pl.BlockSpec((MM_BLK, D_IN), lambda i: (i, 0)),
            pl.BlockSpec((D_IN, D_OUT), lambda i: (0, 0)),
        ],
        out_specs=pl.BlockSpec((MM_BLK, D_OUT), lambda i: (i, 0)),
        out_shape=jax.ShapeDtypeStruct((N_PAD, D_OUT), jnp.float32),
    )(x, W)

    h2, dis = pl.pallas_call(
        _scale_tc_body,
        grid=(mm_blocks,),
        in_specs=[
            pl.BlockSpec((MM_BLK, D_OUT), lambda i: (i, 0)),
            pl.BlockSpec((MM_BLK,), lambda i: (i,)),
            pl.BlockSpec((MM_BLK,), lambda i: (i + mm_blocks,)),
        ],
        out_specs=[
            pl.BlockSpec((MM_BLK, D_OUT), lambda i: (i, 0)),
            pl.BlockSpec((MM_BLK,), lambda i: (i,)),
        ],
        out_shape=[
            jax.ShapeDtypeStruct((N_PAD, D_OUT), jnp.float32),
            jax.ShapeDtypeStruct((N_PAD,), jnp.float32),
        ],
    )(g, degp, degp)
    disr = jnp.broadcast_to(
        dis.reshape(N_PAD // 2, 2, 1), (N_PAD // 2, 2, D_OUT)
    ).reshape(N_PAD // 2, 2 * D_OUT)

    scat_k = functools.partial(
        pl.kernel,
        out_type=jax.ShapeDtypeStruct((NC, N_PAD, D_OUT), jnp.float32),
        mesh=mesh,
        compiler_params=pltpu.CompilerParams(use_tc_tiling_on_sc=False),
        scratch_types=(
            [
                pltpu.VMEM((SCAT_K0, CHUNK), jnp.int32),
                pltpu.VMEM((SCAT_K0, CHUNK), jnp.int32),
            ]
            + [pltpu.VMEM((CHUNK, D_OUT), jnp.float32)] * 3
            + [
                pltpu.VMEM_SHARED((N_PAD, D_OUT), jnp.float32),
                pltpu.VMEM_SHARED((N_PAD, D_OUT), jnp.float32),
            ]
            + [pltpu.SemaphoreType.DMA] * 6
        ),
    )(_scatter_body)
    accp = scat_k(ei3.reshape(2, NW * CPT, CHUNK), h2)
    ap = accp.reshape(NC, N_PAD // 2, 2 * D_OUT)

    F_BLK = 512
    out_blocks = -(-(N_NODES // 2) // F_BLK)
    outp = pl.pallas_call(
        _final_tc_body,
        grid=(out_blocks,),
        in_specs=[
            pl.BlockSpec((1, F_BLK, 2 * D_OUT), lambda i: (0, i, 0)),
            pl.BlockSpec((1, F_BLK, 2 * D_OUT), lambda i: (1, i, 0)),
            pl.BlockSpec((F_BLK, 2 * D_OUT), lambda i: (i, 0)),
            pl.BlockSpec((1, 2 * D_OUT), lambda i: (0, 0)),
        ],
        out_specs=pl.BlockSpec((F_BLK, 2 * D_OUT), lambda i: (i, 0)),
        out_shape=jax.ShapeDtypeStruct((N_NODES // 2, 2 * D_OUT), jnp.float32),
    )(ap, ap, disr, b2)

    return outp.reshape(N_NODES, D_OUT)


# speedup vs baseline: 3.0643x; 1.0141x over previous
import functools

import jax
import jax.numpy as jnp
from jax import lax
from jax.experimental import pallas as pl
from jax.experimental.pallas import tpu as pltpu
from jax.experimental.pallas import tpu_sc as plsc

N_NODES = 10000
N_EDGES = 320000
D_IN = 128
D_OUT = 64

NC, NS, L = 2, 16, 16
NW = NC * NS
CHUNK = 128
CPT = 2 * (-(-N_EDGES // (NW * CHUNK * 2)))
E_PAD = NW * CPT * CHUNK
ROW_BLK = 512
N_PAD = ROW_BLK * (-(-(N_NODES + 1) // ROW_BLK))
RPT = N_PAD // NS


DEG_K0 = 104
DEG_K1 = NW * CPT // NS - DEG_K0


def _deg_body(ei_hbm, degp_hbm, dst_all, ones_v, zcol_v, deg_sh, ssem):
    c = lax.axis_index("c")
    s = lax.axis_index("s")
    row0 = s * RPT

    for i in range(CHUNK // L):
        ones_v[pl.ds(i * L, L)] = jnp.ones((L,), jnp.float32)

    def zero(i, carry):
        zcol_v[pl.ds(i * L, L)] = jnp.zeros((L,), jnp.float32)
        return carry

    lax.fori_loop(0, RPT // L, zero, 0)
    pltpu.sync_copy(zcol_v, deg_sh.at[pl.ds(row0, RPT)])
    plsc.subcore_barrier()

    def run(base, k):
        pltpu.sync_copy(ei_hbm.at[1, pl.ds(base, k)], dst_all.at[pl.ds(0, k)])

        def body(j, carry):
            pltpu.async_copy(ones_v, deg_sh.at[dst_all.at[j]], ssem, add=True)
            return carry

        lax.fori_loop(0, k, body, 0)

        def drain(j, carry):
            pltpu.make_async_copy(ones_v, deg_sh.at[dst_all.at[0]],
                                  ssem).wait()
            return carry

        lax.fori_loop(0, k, drain, 0)

    @pl.when(c == 0)
    def _():
        run(s * DEG_K0, DEG_K0)

    @pl.when(c == 1)
    def _():
        run(NS * DEG_K0 + s * DEG_K1, DEG_K1)

    plsc.subcore_barrier()
    pltpu.sync_copy(deg_sh.at[pl.ds(row0, RPT)],
                    degp_hbm.at[pl.ds(c * N_PAD + row0, RPT)])


SCAT_K0 = 83
SCAT_K1 = NW * CPT // NS - SCAT_K0


def _scatter_body(ei_hbm, h2_hbm, accp_hbm,
                  src_all, dst_all, rows0, rows1, rows2,
                  h2_sh, acc_sh,
                  gsem0, gsem1, gsem2, ssem0, ssem1, ssem2):
    c = lax.axis_index("c")
    s = lax.axis_index("s")
    row0 = s * RPT

    pltpu.sync_copy(h2_hbm.at[pl.ds(row0, RPT)], h2_sh.at[pl.ds(row0, RPT)])

    @pl.when(c == 0)
    def _():
        pltpu.sync_copy(h2_hbm.at[pl.ds(row0, RPT)], acc_sh.at[pl.ds(row0, RPT)])

    @pl.when(c == 1)
    def _():
        def zero(i, carry):
            for k in range(D_OUT // L):
                rows0[i, pl.ds(k * L, L)] = jnp.zeros((L,), jnp.float32)
            return carry

        lax.fori_loop(0, CHUNK, zero, 0)
        for j in range(RPT // CHUNK):
            pltpu.sync_copy(rows0, acc_sh.at[pl.ds(row0 + j * CHUNK, CHUNK)])

    plsc.subcore_barrier()

    rows = (rows0, rows1, rows2)
    gsem = (gsem0, gsem1, gsem2)
    ssem = (ssem0, ssem1, ssem2)

    def pipeline(base, k, srcs):
        pltpu.sync_copy(ei_hbm.at[0, pl.ds(base, k)], src_all.at[pl.ds(0, k)])
        pltpu.sync_copy(ei_hbm.at[1, pl.ds(base, k)], dst_all.at[pl.ds(0, k)])

        def gather(j, b):
            pltpu.async_copy(srcs[b].at[src_all.at[j]], rows[b], gsem[b])

        def gwait(b):
            pltpu.make_async_copy(srcs[b].at[src_all.at[0]], rows[b],
                                  gsem[b]).wait()

        def sstart(j, b):
            pltpu.async_copy(rows[b], acc_sh.at[dst_all.at[j]], ssem[b],
                             add=True)

        def swait(b):
            pltpu.make_async_copy(rows[b], acc_sh.at[dst_all.at[0]],
                                  ssem[b]).wait()

        gather(0, 0)
        gather(1, 1)
        gwait(0)
        sstart(0, 0)
        gather(2, 2)
        gwait(1)
        sstart(1, 1)

        def body(i, carry):
            for u in range(3):
                j3 = 3 * i + 2 + u
                b = (2 + u) % 3
                swait(u)
                gather(j3 + 1, u)
                gwait(b)
                sstart(j3, b)
            return carry

        lax.fori_loop(0, (k - 5) // 3, body, 0)
        for j in range(k - 3, k):
            swait((j + 1) % 3)
            if j + 1 < k:
                gather(j + 1, (j + 1) % 3)
            gwait(j % 3)
            sstart(j, j % 3)
        swait((k - 2) % 3)
        swait((k - 1) % 3)

    @pl.when(c == 0)
    def _():
        pipeline(s * SCAT_K0, SCAT_K0, (h2_sh, h2_sh, h2_sh))

    @pl.when(c == 1)
    def _():
        pipeline(NS * SCAT_K0 + s * SCAT_K1, SCAT_K1, (h2_sh, h2_sh, h2_sh))

    plsc.subcore_barrier()
    pltpu.sync_copy(acc_sh.at[pl.ds(row0, RPT)],
                    accp_hbm.at[c, pl.ds(row0, RPT)])


def _mm_tc_body(x_ref, w_ref, g_ref):
    g_ref[...] = jnp.dot(x_ref[...], w_ref[...],
                         preferred_element_type=jnp.float32)


def _scale_tc_body(g_ref, deg0_ref, deg1_ref, h2_ref, dis_ref):
    deg = deg0_ref[...] + deg1_ref[...] + 1.0
    dis = lax.rsqrt(deg)
    h2_ref[...] = g_ref[...] * dis[:, None]
    dis_ref[...] = dis


def _final_tc_body(a0_ref, a1_ref, disr_ref, b_ref, out_ref):
    a = a0_ref[0] + a1_ref[0]
    out_ref[...] = a * disr_ref[...] + b_ref[...]


def kernel(x, edge_index, W, b):
    pad = E_PAD - N_EDGES
    ei3 = jnp.pad(edge_index.astype(jnp.int32), ((0, 0), (0, pad)),
                  constant_values=N_NODES).reshape(2, NW, CPT, CHUNK)
    b2 = jnp.concatenate([b, b]).reshape(1, 2 * D_OUT)

    mesh = plsc.VectorSubcoreMesh(core_axis_name="c", subcore_axis_name="s")

    deg_k = functools.partial(
        pl.kernel,
        out_type=jax.ShapeDtypeStruct((NC * N_PAD,), jnp.float32),
        mesh=mesh,
        scratch_types=[
            pltpu.VMEM((DEG_K0, CHUNK), jnp.int32),
            pltpu.VMEM((CHUNK,), jnp.float32),
            pltpu.VMEM((RPT,), jnp.float32),
            pltpu.VMEM_SHARED((N_PAD,), jnp.float32),
            pltpu.SemaphoreType.DMA,
        ],
    )(_deg_body)
    degp = deg_k(ei3.reshape(2, NW * CPT, CHUNK))

    MM_BLK = 2048
    mm_blocks = N_PAD // MM_BLK
    g = pl.pallas_call(
        _mm_tc_body,
        grid=(mm_blocks,),
        in_specs=[
            pl.BlockSpec((MM_BLK, D_IN), lambda i: (i, 0)),
            pl.BlockSpec((D_IN, D_OUT), lambda i: (0, 0)),
        ],
        out_specs=pl.BlockSpec((MM_BLK, D_OUT), lambda i: (i, 0)),
        out_shape=jax.ShapeDtypeStruct((N_PAD, D_OUT), jnp.float32),
    )(x, W)

    h2, dis = pl.pallas_call(
        _scale_tc_body,
        grid=(mm_blocks,),
        in_specs=[
            pl.BlockSpec((MM_BLK, D_OUT), lambda i: (i, 0)),
            pl.BlockSpec((MM_BLK,), lambda i: (i,)),
            pl.BlockSpec((MM_BLK,), lambda i: (i + mm_blocks,)),
        ],
        out_specs=[
            pl.BlockSpec((MM_BLK, D_OUT), lambda i: (i, 0)),
            pl.BlockSpec((MM_BLK,), lambda i: (i,)),
        ],
        out_shape=[
            jax.ShapeDtypeStruct((N_PAD, D_OUT), jnp.float32),
            jax.ShapeDtypeStruct((N_PAD,), jnp.float32),
        ],
    )(g, degp, degp)
    disr = jnp.broadcast_to(
        dis.reshape(N_PAD // 2, 2, 1), (N_PAD // 2, 2, D_OUT)
    ).reshape(N_PAD // 2, 2 * D_OUT)

    scat_k = functools.partial(
        pl.kernel,
        out_type=jax.ShapeDtypeStruct((NC, N_PAD, D_OUT), jnp.float32),
        mesh=mesh,
        compiler_params=pltpu.CompilerParams(use_tc_tiling_on_sc=False),
        scratch_types=(
            [
                pltpu.VMEM((SCAT_K0, CHUNK), jnp.int32),
                pltpu.VMEM((SCAT_K0, CHUNK), jnp.int32),
            ]
            + [pltpu.VMEM((CHUNK, D_OUT), jnp.float32)] * 3
            + [
                pltpu.VMEM_SHARED((N_PAD, D_OUT), jnp.float32),
                pltpu.VMEM_SHARED((N_PAD, D_OUT), jnp.float32),
            ]
            + [pltpu.SemaphoreType.DMA] * 6
        ),
    )(_scatter_body)
    accp = scat_k(ei3.reshape(2, NW * CPT, CHUNK), h2)
    ap = accp.reshape(NC, N_PAD // 2, 2 * D_OUT)

    F_BLK = 1024
    out_blocks = -(-(N_NODES // 2) // F_BLK)
    outp = pl.pallas_call(
        _final_tc_body,
        grid=(out_blocks,),
        in_specs=[
            pl.BlockSpec((1, F_BLK, 2 * D_OUT), lambda i: (0, i, 0)),
            pl.BlockSpec((1, F_BLK, 2 * D_OUT), lambda i: (1, i, 0)),
            pl.BlockSpec((F_BLK, 2 * D_OUT), lambda i: (i, 0)),
            pl.BlockSpec((1, 2 * D_OUT), lambda i: (0, 0)),
        ],
        out_specs=pl.BlockSpec((F_BLK, 2 * D_OUT), lambda i: (i, 0)),
        out_shape=jax.ShapeDtypeStruct((N_NODES // 2, 2 * D_OUT), jnp.float32),
    )(ap, ap, disr, b2)

    return outp.reshape(N_NODES, D_OUT)
